# baseline (speedup 1.0000x reference)
import functools

import jax
import jax.numpy as jnp
from jax import lax
from jax.experimental import pallas as pl
from jax.experimental.pallas import tpu as pltpu
from jax.experimental.pallas import tpu_sc as plsc

SIZE = 100000
INPUT_SIZE = 128
CH = 160
NCHUNK = SIZE // CH
NW = 32
TMAX = (NCHUNK + NW - 1) // NW
G = CH // 16

_mesh = plsc.VectorSubcoreMesh(
    core_axis_name="c", subcore_axis_name="s", num_cores=2, num_subcores=16)


@functools.partial(
    pl.kernel,
    out_type=[
        jax.ShapeDtypeStruct((SIZE,), jnp.float32),
        jax.ShapeDtypeStruct((NW, 16), jnp.int32),
    ],
    mesh=_mesh,
    scratch_types=[
        pltpu.VMEM((CH, INPUT_SIZE), jnp.float32),
        pltpu.VMEM((CH, INPUT_SIZE), jnp.float32),
        pltpu.VMEM((CH,), jnp.float32),
        pltpu.VMEM((1, INPUT_SIZE), jnp.float32),
        pltpu.VMEM((16,), jnp.int32),
        pltpu.SemaphoreType.DMA,
        pltpu.SemaphoreType.DMA,
    ],
    compiler_params=pltpu.CompilerParams(needs_layout_passes=False),
)
def _sc_matvec(inp_hbm, conn_hbm, out_hbm, bests_hbm,
               buf0, buf1, obuf, minp, bestv, sem0, sem1):
    wid = lax.axis_index("s") * 2 + lax.axis_index("c")
    pltpu.sync_copy(inp_hbm, minp)

    lane = lax.iota(jnp.int32, 16)
    zero16 = jnp.zeros((16,), jnp.int32)
    bestv[...] = jnp.full((16,), jnp.int32(-2**31 + 1), jnp.int32)

    bufs = (buf0, buf1)
    sems = (sem0, sem1)

    def start(t, buf, sem):
        chunk = wid + t * NW

        @pl.when(chunk < NCHUNK)
        def _():
            pltpu.async_copy(conn_hbm.at[pl.ds(chunk * CH, CH)], buf, sem)

    def process(t, buf, sem):
        chunk = wid + t * NW

        @pl.when(chunk < NCHUNK)
        def _():
            pltpu.make_async_copy(conn_hbm.at[pl.ds(chunk * CH, CH)],
                                  buf, sem).wait()

            def col_body(j, accs):
                colv = (j + lane) & (INPUT_SIZE - 1)
                sv = plsc.load_gather(minp, [zero16, colv])
                new = []
                for g in range(G):
                    v = plsc.load_gather(buf, [g * 16 + lane, colv])
                    new.append(accs[g] + v * sv)
                return tuple(new)

            accs = lax.fori_loop(
                0, INPUT_SIZE, col_body,
                tuple(jnp.zeros((16,), jnp.float32) for _ in range(G)),
                unroll=8)

            best = bestv[...]
            for g in range(G):
                obuf[pl.ds(g * 16, 16)] = accs[g]
                rows = chunk * CH + g * 16 + lane
                key = (accs[g].astype(jnp.int32) << 17) | (131071 - rows)
                best = jnp.maximum(best, key)
            bestv[...] = best
            pltpu.sync_copy(obuf, out_hbm.at[pl.ds(chunk * CH, CH)])

    start(0, buf0, sem0)
    start(1, buf1, sem1)

    def pair_body(i, carry):
        t = 2 * i
        process(t, buf0, sem0)
        start(t + 2, buf0, sem0)
        process(t + 1, buf1, sem1)
        start(t + 3, buf1, sem1)
        return carry

    lax.fori_loop(0, TMAX // 2, pair_body, jnp.int32(0))

    pltpu.sync_copy(bestv, bests_hbm.at[wid])


def kernel(input_array, connected):
    inp = input_array.astype(jnp.float32).reshape(1, INPUT_SIZE)
    overlaps, bests = _sc_matvec(inp, connected)
    winner = 131071 - (jnp.max(bests) & 131071)
    return overlaps, winner


# speedup vs baseline: 1.5833x; 1.5833x over previous
import functools

import jax
import jax.numpy as jnp
from jax import lax
from jax.experimental import pallas as pl
from jax.experimental.pallas import tpu as pltpu
from jax.experimental.pallas import tpu_sc as plsc

SIZE = 100000
INPUT_SIZE = 128

SC_CH = 160
SC_NCHUNK = 160
SC_ROWS = SC_NCHUNK * SC_CH
TC_ROWS = SIZE - SC_ROWS
SC_BASE = TC_ROWS

CH = 12800
TC_CHUNKS = [(k * CH, CH) for k in range(5)] + [(5 * CH, TC_ROWS - 5 * CH)]

NW = 32
TMAX = (SC_NCHUNK + NW - 1) // NW
TSLOTS = ((TMAX + 1) // 2) * 2
G = SC_CH // 16

_mesh = plsc.VectorSubcoreMesh(
    core_axis_name="c", subcore_axis_name="s", num_cores=2, num_subcores=16)


@functools.partial(
    pl.kernel,
    out_type=[
        jax.ShapeDtypeStruct((SC_ROWS,), jnp.float32),
        jax.ShapeDtypeStruct((NW, 16), jnp.int32),
    ],
    mesh=_mesh,
    scratch_types=[
        pltpu.VMEM((SC_CH, INPUT_SIZE), jnp.float32),
        pltpu.VMEM((SC_CH, INPUT_SIZE), jnp.float32),
        pltpu.VMEM((SC_CH,), jnp.float32),
        pltpu.VMEM((1, INPUT_SIZE), jnp.float32),
        pltpu.VMEM((16,), jnp.int32),
        pltpu.SemaphoreType.DMA,
        pltpu.SemaphoreType.DMA,
    ],
    compiler_params=pltpu.CompilerParams(needs_layout_passes=False),
)
def _sc_matvec(inp_hbm, conn_hbm, out_hbm, bests_hbm,
               buf0, buf1, obuf, minp, bestv, sem0, sem1):
    wid = lax.axis_index("s") * 2 + lax.axis_index("c")
    pltpu.sync_copy(inp_hbm, minp)

    lane = lax.iota(jnp.int32, 16)
    zero16 = jnp.zeros((16,), jnp.int32)
    bestv[...] = jnp.full((16,), jnp.int32(-2**31 + 1), jnp.int32)

    def start(t, buf, sem):
        chunk = wid + t * NW

        @pl.when(chunk < SC_NCHUNK)
        def _():
            pltpu.async_copy(
                conn_hbm.at[pl.ds(SC_BASE + chunk * SC_CH, SC_CH)], buf, sem)

    def process(t, buf, sem):
        chunk = wid + t * NW

        @pl.when(chunk < SC_NCHUNK)
        def _():
            pltpu.make_async_copy(
                conn_hbm.at[pl.ds(SC_BASE + chunk * SC_CH, SC_CH)],
                buf, sem).wait()

            def col_body(j, accs):
                colv = (j + lane) & (INPUT_SIZE - 1)
                sv = plsc.load_gather(minp, [zero16, colv])
                new = []
                for g in range(G):
                    v = plsc.load_gather(buf, [g * 16 + lane, colv])
                    new.append(accs[g] + v * sv)
                return tuple(new)

            accs = lax.fori_loop(
                0, INPUT_SIZE, col_body,
                tuple(jnp.zeros((16,), jnp.float32) for _ in range(G)),
                unroll=8)

            best = bestv[...]
            for g in range(G):
                obuf[pl.ds(g * 16, 16)] = accs[g]
                rows = SC_BASE + chunk * SC_CH + g * 16 + lane
                key = (accs[g].astype(jnp.int32) << 17) | (131071 - rows)
                best = jnp.maximum(best, key)
            bestv[...] = best
            pltpu.sync_copy(obuf, out_hbm.at[pl.ds(chunk * SC_CH, SC_CH)])

    start(0, buf0, sem0)
    start(1, buf1, sem1)

    def pair_body(i, carry):
        t = 2 * i
        process(t, buf0, sem0)
        start(t + 2, buf0, sem0)
        process(t + 1, buf1, sem1)
        start(t + 3, buf1, sem1)
        return carry

    lax.fori_loop(0, TSLOTS // 2, pair_body, jnp.int32(0))

    pltpu.sync_copy(bestv, bests_hbm.at[wid])


def _tc_body(inp_ref, conn_ref, out_ref, win_ref,
             b0, b1, o0, o1, o2, best_ref, si0, si1, so0, so1, so2):
    bufs = (b0, b1)
    isems = (si0, si1)
    LAST = len(TC_CHUNKS) - 1
    obufs = [o0, o1] * 3
    osems = [so0, so1] * 3
    obufs[LAST] = o2
    osems[LAST] = so2
    inp = inp_ref[...].astype(jnp.float32)
    best_ref[0] = jnp.int32(-2**31 + 1)

    def start(k):
        row0, n = TC_CHUNKS[k]
        pltpu.async_copy(conn_ref.at[pl.ds(row0, n)],
                         bufs[k % 2].at[pl.ds(0, n)], isems[k % 2])

    start(0)
    start(1)

    for k, (row0, n) in enumerate(TC_CHUNKS):
        pltpu.make_async_copy(conn_ref.at[pl.ds(row0, n)],
                              bufs[k % 2].at[pl.ds(0, n)], isems[k % 2]).wait()
        ov = lax.dot_general(inp, bufs[k % 2][...], (((1,), (1,)), ((), ())),
                             preferred_element_type=jnp.float32)

        flat = row0 + lax.broadcasted_iota(jnp.int32, (1, CH), 1)
        key = (ov.astype(jnp.int32) << 17) | (131071 - flat)
        if n < CH:
            key = jnp.where(flat < TC_ROWS, key, jnp.int32(-2**31 + 1))
        best_ref[0] = jnp.maximum(best_ref[0], jnp.max(key))

        if 2 <= k <= LAST - 1:
            pr, pn = TC_CHUNKS[k - 2]
            pltpu.make_async_copy(obufs[k - 2],
                                  out_ref.at[:, pl.ds(pr, pn)],
                                  osems[k - 2]).wait()
        obufs[k][...] = ov[:, :n]
        pltpu.async_copy(obufs[k], out_ref.at[:, pl.ds(row0, n)], osems[k])
        if k + 2 < len(TC_CHUNKS):
            start(k + 2)

    for k in (LAST - 2, LAST - 1, LAST):
        row0, n = TC_CHUNKS[k]
        pltpu.make_async_copy(obufs[k], out_ref.at[:, pl.ds(row0, n)],
                              osems[k]).wait()

    win_ref[0] = best_ref[0]


def kernel(input_array, connected):
    inp = input_array.astype(jnp.float32).reshape(1, INPUT_SIZE)

    sc_out, sc_bests = _sc_matvec(inp, connected)

    tc_out2d, tc_best1 = pl.pallas_call(
        _tc_body,
        in_specs=[
            pl.BlockSpec((1, INPUT_SIZE), lambda: (0, 0)),
            pl.BlockSpec(memory_space=pltpu.HBM),
        ],
        out_specs=[
            pl.BlockSpec(memory_space=pltpu.HBM),
            pl.BlockSpec(memory_space=pltpu.SMEM),
        ],
        out_shape=[
            jax.ShapeDtypeStruct((1, TC_ROWS), jnp.float32),
            jax.ShapeDtypeStruct((1,), jnp.int32),
        ],
        scratch_shapes=[
            pltpu.VMEM((CH, INPUT_SIZE), jnp.float32),
            pltpu.VMEM((CH, INPUT_SIZE), jnp.float32),
            pltpu.VMEM((1, CH), jnp.float32),
            pltpu.VMEM((1, CH), jnp.float32),
            pltpu.VMEM((1, TC_ROWS - 5 * CH), jnp.float32),
            pltpu.SMEM((1,), jnp.int32),
            pltpu.SemaphoreType.DMA,
            pltpu.SemaphoreType.DMA,
            pltpu.SemaphoreType.DMA,
            pltpu.SemaphoreType.DMA,
            pltpu.SemaphoreType.DMA,
        ],
    )(inp, connected)

    overlaps = jnp.concatenate([tc_out2d.reshape(TC_ROWS), sc_out])
    best = jnp.maximum(tc_best1[0], jnp.max(sc_bests))
    winner = 131071 - (best & 131071)
    return overlaps, winner
